# Initial kernel scaffold; baseline (speedup 1.0000x reference)
import jax
import jax.numpy as jnp
from jax.experimental import pallas as pl


def kernel(x, e, edge_index, psi_W1, psi_b1, psi_W2, psi_b2, phi_W1, phi_b1, phi_W2, phi_b2, nu_W1, nu_b1, nu_W2, nu_b2):
    raise NotImplementedError("write your pallas kernel here")


# speedup vs baseline: 1.0859x; 1.0859x over previous
import functools

import jax
import jax.numpy as jnp
from jax import lax
from jax.experimental import pallas as pl
from jax.experimental.pallas import tpu as pltpu
from jax.experimental.pallas import tpu_sc as plsc

N = 10000
NP = 10240
E = 160000
XD = 256
ED = 16
H = 256
GW = 512

NC, NS = 2, 16
NW = NC * NS
EPW = E // NW
KG = 40
GCH = EPW // KG

CPW = H // NW
KS = 640
SCH = E // KS


def _tables_body(x_ref, wa_ref, wb_ref, ba_ref, a_ref, b_ref):
    xb = x_ref[...]
    a_ref[...] = jnp.dot(xb, wa_ref[...], preferred_element_type=jnp.float32) + ba_ref[...]
    b_ref[...] = jnp.dot(xb, wb_ref[...], preferred_element_type=jnp.float32)


def _tables(x2, WA, WB, bA):
    BN = 2048
    return pl.pallas_call(
        _tables_body,
        grid=(NP // BN,),
        in_specs=[
            pl.BlockSpec((BN, XD), lambda i: (i, 0)),
            pl.BlockSpec((XD, GW), lambda i: (0, 0)),
            pl.BlockSpec((XD, GW), lambda i: (0, 0)),
            pl.BlockSpec((1, GW), lambda i: (0, 0)),
        ],
        out_specs=[
            pl.BlockSpec((BN, GW), lambda i: (i, 0)),
            pl.BlockSpec((BN, GW), lambda i: (i, 0)),
        ],
        out_shape=[jax.ShapeDtypeStruct((NP, GW), jnp.float32)] * 2,
    )(x2, WA, WB, bA)


def _sc_gather(A, B, recv_flat, send_flat):
    mesh = plsc.VectorSubcoreMesh(core_axis_name="c", subcore_axis_name="s")

    @functools.partial(
        pl.kernel,
        mesh=mesh,
        out_type=jax.ShapeDtypeStruct((NW, EPW, GW), jnp.float32),
        scratch_types=[
            pltpu.VMEM((KG,), jnp.int32),
            pltpu.VMEM((KG,), jnp.int32),
            pltpu.VMEM((KG, GW), jnp.float32),
            pltpu.VMEM((KG, GW), jnp.float32),
            pltpu.SemaphoreType.DMA,
            pltpu.SemaphoreType.DMA,
        ],
    )
    def k(a_hbm, b_hbm, recv_hbm, send_hbm, out_hbm, idxr, idxs, bufa, bufb,
          sema, semb):
        wid = lax.axis_index("s") * NC + lax.axis_index("c")

        def chunk(ci, carry):
            off = ci * KG
            pltpu.sync_copy(recv_hbm.at[pl.ds(wid * EPW + off, KG)], idxr)
            pltpu.sync_copy(send_hbm.at[pl.ds(wid * EPW + off, KG)], idxs)
            ca = pltpu.async_copy(a_hbm.at[idxr], bufa, sema)
            cb = pltpu.async_copy(b_hbm.at[idxs], bufb, semb)
            ca.wait()
            cb.wait()

            def add_row(i, c2):
                for v in range(GW // 16):
                    sl = pl.ds(v * 16, 16)
                    bufa[i, sl] = bufa[i, sl] + bufb[i, sl]
                return c2

            lax.fori_loop(0, KG, add_row, 0)
            pltpu.sync_copy(bufa, out_hbm.at[wid, pl.ds(off, KG)])
            return carry

        lax.fori_loop(0, GCH, chunk, 0)

    return k(A, B, recv_flat, send_flat)


def _edge_body(g_ref, e_ref, we_ref, pw2_ref, pb2_ref, f_ref, fw2_ref,
               fb2_ref, eout_ref, mt_ref):
    g = g_ref[...]
    h1 = jnp.maximum(
        g[:, :H] + jnp.dot(e_ref[...], we_ref[...], preferred_element_type=jnp.float32),
        0.0)
    eo = jnp.dot(h1, pw2_ref[...], preferred_element_type=jnp.float32) + pb2_ref[...]
    eout_ref[...] = eo
    h2 = jnp.maximum(
        g[:, H:] + jnp.dot(eo, f_ref[...], preferred_element_type=jnp.float32),
        0.0)
    mt = lax.dot_general(fw2_ref[...], h2, (((0,), (1,)), ((), ())),
                         preferred_element_type=jnp.float32)
    mt_ref[...] = mt + fb2_ref[...]


def _edge_mlp(G, e2, We, pW2, pb2, F, fW2, fb2col):
    TE = 1280
    return pl.pallas_call(
        _edge_body,
        grid=(E // TE,),
        in_specs=[
            pl.BlockSpec((TE, GW), lambda i: (i, 0)),
            pl.BlockSpec((TE, ED), lambda i: (i, 0)),
            pl.BlockSpec((ED, H), lambda i: (0, 0)),
            pl.BlockSpec((H, ED), lambda i: (0, 0)),
            pl.BlockSpec((1, ED), lambda i: (0, 0)),
            pl.BlockSpec((ED, H), lambda i: (0, 0)),
            pl.BlockSpec((H, H), lambda i: (0, 0)),
            pl.BlockSpec((H, 1), lambda i: (0, 0)),
        ],
        out_specs=[
            pl.BlockSpec((TE, ED), lambda i: (i, 0)),
            pl.BlockSpec((H, TE), lambda i: (0, i)),
        ],
        out_shape=[
            jax.ShapeDtypeStruct((E, ED), jnp.float32),
            jax.ShapeDtypeStruct((H, E), jnp.float32),
        ],
    )(G, e2, We, pW2, pb2, F, fW2, fb2col)


def _sc_scatter(mT, recv_flat):
    mesh = plsc.VectorSubcoreMesh(core_axis_name="c", subcore_axis_name="s")

    @functools.partial(
        pl.kernel,
        mesh=mesh,
        out_type=jax.ShapeDtypeStruct((H * NP,), jnp.float32),
        compiler_params=pltpu.CompilerParams(needs_layout_passes=False),
        scratch_types=[
            pltpu.VMEM((KS,), jnp.int32),
            pltpu.VMEM((CPW, KS), jnp.float32),
            pltpu.VMEM((CPW * NP,), jnp.float32),
            pltpu.SemaphoreType.DMA,
        ],
    )
    def k(mt_hbm, r_hbm, aggt_hbm, idxb, mbuf, acc, sem):
        wid = lax.axis_index("s") * NC + lax.axis_index("c")
        zeros = jnp.zeros((16,), jnp.float32)

        def zchunk(j, c2):
            for v in range(16):
                acc[pl.ds((j * 16 + v) * 16, 16)] = zeros
            return c2

        lax.fori_loop(0, CPW * NP // 256, zchunk, 0)

        def chunk(ci, carry):
            off = ci * KS
            pltpu.sync_copy(r_hbm.at[pl.ds(off, KS)], idxb)
            cm = pltpu.async_copy(
                mt_hbm.at[pl.ds(wid * CPW, CPW), pl.ds(off, KS)], mbuf, sem)
            cm.wait()

            def group(g, c2):
                ids = idxb[pl.ds(g * 16, 16)]
                for c in range(CPW):
                    data = mbuf[c, pl.ds(g * 16, 16)]
                    plsc.addupdate_scatter(acc, [ids + (c * NP)], data)
                return c2

            lax.fori_loop(0, KS // 16, group, 0)
            return carry

        lax.fori_loop(0, SCH, chunk, 0)
        pltpu.sync_copy(acc, aggt_hbm.at[pl.ds(wid * CPW * NP, CPW * NP)])

    return k(mT, recv_flat)


def _node_body(x_ref, aggt_ref, na_ref, nb_ref, b1_ref, w2_ref, b2_ref,
               out_ref):
    agg_contrib = lax.dot_general(aggt_ref[...], nb_ref[...],
                                  (((0,), (0,)), ((), ())),
                                  preferred_element_type=jnp.float32)
    h = jnp.maximum(
        jnp.dot(x_ref[...], na_ref[...], preferred_element_type=jnp.float32)
        + agg_contrib + b1_ref[...], 0.0)
    out_ref[...] = jnp.dot(h, w2_ref[...], preferred_element_type=jnp.float32) + b2_ref[...]


def _node_mlp(x2, aggT, nuA, nuB, nb1, nW2, nb2):
    BN = 2048
    return pl.pallas_call(
        _node_body,
        grid=(NP // BN,),
        in_specs=[
            pl.BlockSpec((BN, XD), lambda i: (i, 0)),
            pl.BlockSpec((H, BN), lambda i: (0, i)),
            pl.BlockSpec((XD, H), lambda i: (0, 0)),
            pl.BlockSpec((H, H), lambda i: (0, 0)),
            pl.BlockSpec((1, H), lambda i: (0, 0)),
            pl.BlockSpec((H, XD), lambda i: (0, 0)),
            pl.BlockSpec((1, XD), lambda i: (0, 0)),
        ],
        out_specs=pl.BlockSpec((BN, XD), lambda i: (i, 0)),
        out_shape=jax.ShapeDtypeStruct((NP, XD), jnp.float32),
    )(x2, aggT, nuA, nuB, nb1, nW2, nb2)


def kernel(x, e, edge_index, psi_W1, psi_b1, psi_W2, psi_b2,
           phi_W1, phi_b1, phi_W2, phi_b2, nu_W1, nu_b1, nu_W2, nu_b2):
    x2 = jnp.pad(x.reshape(N, XD), ((0, NP - N), (0, 0)))
    e2 = e.reshape(E, ED)
    ei = edge_index.astype(jnp.int32)
    send_flat = ei[0]
    recv_flat = ei[1]

    We = psi_W1[:ED]
    WA = jnp.concatenate([psi_W1[ED:ED + XD], phi_W1[:XD]], axis=1)
    WB = jnp.concatenate([psi_W1[ED + XD:], phi_W1[XD:2 * XD]], axis=1)
    bA = jnp.concatenate([psi_b1, phi_b1]).reshape(1, GW)
    F = phi_W1[2 * XD:]

    A, B = _tables(x2, WA, WB, bA)
    G = _sc_gather(A, B, recv_flat, send_flat).reshape(E, GW)
    e_out, mT = _edge_mlp(G, e2, We, psi_W2, psi_b2.reshape(1, ED),
                          F, phi_W2, phi_b2.reshape(H, 1))
    aggT = _sc_scatter(mT, recv_flat).reshape(H, NP)
    x_out = _node_mlp(x2, aggT, nu_W1[:XD], nu_W1[XD:], nu_b1.reshape(1, H),
                      nu_W2, nu_b2.reshape(1, XD))
    return (x_out[:N].reshape(1, N, XD), e_out.reshape(1, E, ED))
